# Initial kernel scaffold; baseline (speedup 1.0000x reference)
import jax
import jax.numpy as jnp
from jax.experimental import pallas as pl


def kernel(X, edge_index, edge_weight, W1, as1, ad1, b1, W2, as2, ad2, b2, W3, as3, ad3, b3, Wl, bl):
    raise NotImplementedError("write your pallas kernel here")


# speedup vs baseline: 15.5134x; 15.5134x over previous
import functools

import jax
import jax.numpy as jnp
from jax import lax
from jax.experimental import pallas as pl
from jax.experimental.pallas import tpu as pltpu
from jax.experimental.pallas import tpu_sc as plsc

NC = 2
NS = 16
NW = NC * NS
C = 128
DH = 64


def _proj_body(x_ref, w_ref, avs_ref, avd_ref, h0_ref, h1_ref, asrc_ref,
               adst_ref, mx_ref):
    _proj_inner(x_ref[...], w_ref, avs_ref, avd_ref, h0_ref, h1_ref,
                asrc_ref, adst_ref, mx_ref)


def _finalize(o00_ref, o01_ref, o10_ref, o11_ref, d0_ref, d1_ref, b_ref):
    den = d0_ref[...] + d1_ref[...] + 1e-16
    o = jnp.concatenate(
        [o00_ref[...] + o01_ref[...], o10_ref[...] + o11_ref[...]], axis=1)
    o = o / den + b_ref[...]
    return jnp.where(o > 0.0, o, jnp.exp(o) - 1.0)


def _fin_proj_body(o00_ref, o01_ref, o10_ref, o11_ref, d0_ref, d1_ref, b_ref,
                   w_ref, avs_ref, avd_ref, h0_ref, h1_ref, asrc_ref,
                   adst_ref, mx_ref):
    x = _finalize(o00_ref, o01_ref, o10_ref, o11_ref, d0_ref, d1_ref, b_ref)
    _proj_inner(x, w_ref, avs_ref, avd_ref, h0_ref, h1_ref, asrc_ref,
                adst_ref, mx_ref)


def _proj_inner(x, w_ref, avs_ref, avd_ref, h0_ref, h1_ref, asrc_ref,
                adst_ref, mx_ref):
    i = pl.program_id(0)
    h = jnp.dot(x, w_ref[...], preferred_element_type=jnp.float32)
    h0_ref[...] = h[:, :DH]
    h1_ref[...] = h[:, DH:]
    asrc = jnp.sum(h * avs_ref[...], axis=1, keepdims=True)
    adst = jnp.sum(h * avd_ref[...], axis=1, keepdims=True)
    asrc_ref[...] = asrc
    adst_ref[...] = adst
    cur_s = jnp.max(asrc)
    cur_d = jnp.max(adst)
    rows = lax.broadcasted_iota(jnp.int32, (8, 128), 0)
    cur = jnp.where(rows < 4, cur_s, cur_d)

    @pl.when(i == 0)
    def _():
        mx_ref[...] = cur

    @pl.when(i > 0)
    def _():
        mx_ref[...] = jnp.maximum(mx_ref[...], cur)


def _final_body(o00_ref, o01_ref, o10_ref, o11_ref, d0_ref, d1_ref, b_ref,
                wl_ref, bl_ref, y_ref):
    x = _finalize(o00_ref, o01_ref, o10_ref, o11_ref, d0_ref, d1_ref, b_ref)
    z = jnp.dot(x, wl_ref[...], preferred_element_type=jnp.float32)
    z = z[:, 0:1] + bl_ref[...]
    y_ref[...] = jax.nn.sigmoid(z)


def _make_tc_kernels(NP, D, BR):
    G = NP // BR
    f32 = jnp.float32
    row_spec = pl.BlockSpec((BR, D), lambda i: (i, 0))
    half_spec = pl.BlockSpec((BR, DH), lambda i: (i, 0))
    col_spec = pl.BlockSpec((BR, 1), lambda i: (i, 0))
    w_spec = pl.BlockSpec((D, D), lambda i: (0, 0))
    a_spec = pl.BlockSpec((1, D), lambda i: (0, 0))
    mx_spec = pl.BlockSpec((8, 128), lambda i: (0, 0))
    proj_out_shape = [jax.ShapeDtypeStruct((NP, DH), f32),
                      jax.ShapeDtypeStruct((NP, DH), f32),
                      jax.ShapeDtypeStruct((NP, 1), f32),
                      jax.ShapeDtypeStruct((NP, 1), f32),
                      jax.ShapeDtypeStruct((8, 128), f32)]
    proj_out_specs = [half_spec, half_spec, col_spec, col_spec, mx_spec]
    fin_in = [half_spec, half_spec, half_spec, half_spec, col_spec, col_spec,
              a_spec]

    proj = pl.pallas_call(
        _proj_body, grid=(G,),
        in_specs=[row_spec, w_spec, a_spec, a_spec],
        out_shape=proj_out_shape, out_specs=proj_out_specs)

    fin_proj = pl.pallas_call(
        _fin_proj_body, grid=(G,),
        in_specs=fin_in + [w_spec, a_spec, a_spec],
        out_shape=proj_out_shape, out_specs=proj_out_specs)

    final = pl.pallas_call(
        _final_body, grid=(G,),
        in_specs=fin_in + [pl.BlockSpec((D, 128), lambda i: (0, 0)),
                           pl.BlockSpec((1, 1), lambda i: (0, 0))],
        out_shape=jax.ShapeDtypeStruct((NP, 1), f32),
        out_specs=col_spec)

    return proj, fin_proj, final


def _make_sc_edge(NP, GPW):
    f32 = jnp.float32
    i32 = jnp.int32
    RPT = NP // NS
    assert RPT % C == 0
    RW = RPT // C
    WAVES = GPW // 8
    mesh = plsc.VectorSubcoreMesh(core_axis_name="c", subcore_axis_name="s")

    @functools.partial(
        pl.kernel,
        compiler_params=pltpu.CompilerParams(use_tc_tiling_on_sc=False),
        out_type=[jax.ShapeDtypeStruct((NC * NP, DH), f32),
                  jax.ShapeDtypeStruct((NC * NP, DH), f32),
                  jax.ShapeDtypeStruct((NC * NP,), f32)],
        mesh=mesh,
        scratch_types=[
            pltpu.VMEM((GPW, C), i32),
            pltpu.VMEM((GPW, C), i32),
            pltpu.VMEM((GPW, C), f32),
            pltpu.VMEM((8, C), f32),
            pltpu.VMEM((8, C), f32),
            pltpu.VMEM((8, C), f32),
            pltpu.VMEM((8, C), f32),
            pltpu.VMEM((2, C, DH), f32),
            pltpu.VMEM((C,), f32),
            pltpu.VMEM((16,), f32),
            pltpu.VMEM_SHARED((NP, DH), f32),
            pltpu.VMEM_SHARED((NP,), f32),
            pltpu.SemaphoreType.DMA,
            pltpu.SemaphoreType.DMA,
            pltpu.SemaphoreType.DMA,
        ],
    )
    def edge(h0_hbm, h1_hbm, asrc_hbm, adst_hbm, srcg_hbm, dstg_hbm, ewg_hbm,
             m_hbm, o0p_hbm, o1p_hbm, dpart_hbm,
             src_v, dst_v, c_v, ag_v, bg_v, p_v, ew_v, rows_v, zden_v, m_v,
             out_sp, den_sp, sem_a, sem_b, sem_r):
        cid = lax.axis_index("c")
        sid = lax.axis_index("s")
        wid = cid * NS + sid
        g0 = wid * GPW
        r0 = sid * RPT

        def zrow(r, carry):
            for k in range(DH // 16):
                rows_v[0, r, pl.ds(k * 16, 16)] = jnp.zeros((16,), f32)
            return carry
        lax.fori_loop(0, C, zrow, 0)
        for k in range(C // 16):
            zden_v[pl.ds(k * 16, 16)] = jnp.zeros((16,), f32)

        def zcp(r, carry):
            pltpu.sync_copy(rows_v.at[0], out_sp.at[pl.ds(r0 + r * C, C)])
            pltpu.sync_copy(zden_v, den_sp.at[pl.ds(r0 + r * C, C)])
            return carry
        lax.fori_loop(0, RW, zcp, 0)

        pltpu.sync_copy(m_hbm, m_v)
        pltpu.sync_copy(srcg_hbm.at[pl.ds(g0, GPW)], src_v)
        pltpu.sync_copy(dstg_hbm.at[pl.ds(g0, GPW)], dst_v)
        plsc.subcore_barrier()

        def wave(w, carry):
            pltpu.sync_copy(ewg_hbm.at[pl.ds(g0 + w * 8, 8)], ew_v)
            for k in range(8):
                g = w * 8 + k
                pltpu.make_async_copy(asrc_hbm.at[src_v.at[g]], ag_v.at[k],
                                      sem_a).start()
                pltpu.make_async_copy(adst_hbm.at[dst_v.at[g]], bg_v.at[k],
                                      sem_b).start()
            m = m_v[...]
            for k in range(8):
                g = w * 8 + k
                pltpu.make_async_copy(asrc_hbm.at[src_v.at[g]], ag_v.at[k],
                                      sem_a).wait()
                pltpu.make_async_copy(adst_hbm.at[dst_v.at[g]], bg_v.at[k],
                                      sem_b).wait()
                for q in range(C // 16):
                    sl = pl.ds(q * 16, 16)
                    x = ag_v[k, sl] + bg_v[k, sl]
                    e = jnp.maximum(x, 0.2 * x)
                    p = jnp.exp(e - m)
                    p_v[k, sl] = p
                    c_v[g, sl] = p * ew_v[k, sl]
                pltpu.sync_copy(p_v.at[k], den_sp.at[dst_v.at[g]], add=True)
            return carry
        lax.fori_loop(0, WAVES, wave, 0)

        plsc.subcore_barrier()
        pltpu.sync_copy(den_sp.at[pl.ds(r0, RPT)],
                        dpart_hbm.at[pl.ds(cid * NP + r0, RPT)])

        def row_phase(h_hbm, op_hbm):
            pltpu.make_async_copy(h_hbm.at[src_v.at[0]], rows_v.at[0],
                                  sem_r).start()

            def rstep(g, carry):
                b = lax.rem(g, 2)

                @pl.when(g + 1 < GPW)
                def _():
                    pltpu.make_async_copy(h_hbm.at[src_v.at[g + 1]],
                                          rows_v.at[1 - b], sem_r).start()
                pltpu.make_async_copy(h_hbm.at[src_v.at[g]], rows_v.at[b],
                                      sem_r).wait()

                def sblk(q, carry2):
                    c16 = c_v[g, pl.ds(q * 16, 16)]
                    for lane in range(16):
                        s = c16[lane]
                        r = q * 16 + lane
                        for k in range(DH // 16):
                            sl = pl.ds(k * 16, 16)
                            rows_v[b, r, sl] = rows_v[b, r, sl] * s
                    return carry2
                lax.fori_loop(0, C // 16, sblk, 0)
                pltpu.sync_copy(rows_v.at[b], out_sp.at[dst_v.at[g]],
                                add=True)
                return carry
            lax.fori_loop(0, GPW, rstep, 0)

            plsc.subcore_barrier()
            pltpu.sync_copy(out_sp.at[pl.ds(r0, RPT)],
                            op_hbm.at[pl.ds(cid * NP + r0, RPT)])

        row_phase(h0_hbm, o0p_hbm)

        def zrow2(r, carry):
            for k in range(DH // 16):
                rows_v[0, r, pl.ds(k * 16, 16)] = jnp.zeros((16,), f32)
            return carry
        lax.fori_loop(0, C, zrow2, 0)

        def zcp2(r, carry):
            pltpu.sync_copy(rows_v.at[0], out_sp.at[pl.ds(r0 + r * C, C)])
            return carry
        lax.fori_loop(0, RW, zcp2, 0)
        plsc.subcore_barrier()

        row_phase(h1_hbm, o1p_hbm)

    return edge


def _ceil_to(x, m):
    return (x + m - 1) // m * m


def kernel(X, edge_index, edge_weight, W1, as1, ad1, b1, W2, as2, ad2, b2,
           W3, as3, ad3, b3, Wl, bl):
    N, D = X.shape
    E = edge_index.shape[1]
    NP = _ceil_to(N, NS * C)
    BR = NP // 8
    EP = _ceil_to(E, NW * 8 * C)
    GPW = EP // (NW * C)

    proj, fin_proj, final = _make_tc_kernels(NP, D, BR)
    edge = _make_sc_edge(NP, GPW)

    f32 = jnp.float32
    Xp = jnp.pad(X, ((0, NP - N), (0, 0)))
    pe = EP - E
    srcg = jnp.pad(edge_index[0], (0, pe)).reshape(EP // C, C)
    dstg = jnp.pad(edge_index[1], (0, pe),
                   constant_values=N).reshape(EP // C, C)
    ewg = jnp.pad(edge_weight, (0, pe)).reshape(EP // C, C)
    bl2 = bl.reshape(1, 1)
    Wlp = jnp.pad(Wl, ((0, 0), (0, 128 - Wl.shape[1])))

    def attn(mx):
        m = jnp.maximum(mx[0, 0] + mx[7, 0], 0.0)
        return jnp.full((16,), m, f32)

    def sc_args(o0, o1, dn):
        return (o0[:NP], o0[NP:], o1[:NP], o1[NP:],
                dn[:NP].reshape(NP, 1), dn[NP:].reshape(NP, 1))

    h0, h1, asrc, adst, mx = proj(Xp, W1, as1.reshape(1, D), ad1.reshape(1, D))
    o0, o1, dn = edge(h0, h1, asrc.reshape(NP), adst.reshape(NP), srcg, dstg,
                      ewg, attn(mx))

    h0, h1, asrc, adst, mx = fin_proj(
        *sc_args(o0, o1, dn), b1.reshape(1, D), W2, as2.reshape(1, D),
        ad2.reshape(1, D))
    o0, o1, dn = edge(h0, h1, asrc.reshape(NP), adst.reshape(NP), srcg, dstg,
                      ewg, attn(mx))

    h0, h1, asrc, adst, mx = fin_proj(
        *sc_args(o0, o1, dn), b2.reshape(1, D), W3, as3.reshape(1, D),
        ad3.reshape(1, D))
    o0, o1, dn = edge(h0, h1, asrc.reshape(NP), adst.reshape(NP), srcg, dstg,
                      ewg, attn(mx))

    y = final(*sc_args(o0, o1, dn), b3.reshape(1, D), Wlp, bl2)
    return y[:N, 0]
